# Initial kernel scaffold; baseline (speedup 1.0000x reference)
import jax
import jax.numpy as jnp
from jax.experimental import pallas as pl


def kernel(h_ligand, h_pocket, e_lp, e_pl, lp_src, lp_dst, pl_src, pl_dst, lp_graph_id, pl_graph_id, W_lp_src, b_lp_src, W_lp_dst, b_lp_dst, W_lp_edge, b_lp_edge, W_pl_src, b_pl_src, W_pl_dst, b_pl_dst, W_pl_edge, b_pl_edge, W_fc_lp, b_fc_lp, W_fc_pl, b_fc_pl):
    raise NotImplementedError("write your pallas kernel here")


# speedup vs baseline: 2.6317x; 2.6317x over previous
import functools

import jax
import jax.numpy as jnp
from jax import lax
from jax.experimental import pallas as pl
from jax.experimental.pallas import tpu as pltpu
from jax.experimental.pallas import tpu_sc as plsc

_N_NODE = 50000
_E = 800000
_NODE_F = 128
_EDGE_F = 16
_HID = 64
_ROW = _HID + 1
_G = 64

_C = 256
_NCHUNK = _E // _C
_NC = 2
_NS = 16
_NW = _NC * _NS

_NODE_BLK = 2000
_EDGE_BLK = 8000


def _node_proj_body(hl, hp, wls, wps, wld, wpd, bls, bps, bld, bpd,
                    hs_lp, hs_pl, hd_lp, hd_pl):
    l = hl[...]
    p = hp[...]
    hs_lp[...] = jnp.dot(l, wls[...], preferred_element_type=jnp.float32) + bls[...]
    hs_pl[...] = jnp.dot(l, wps[...], preferred_element_type=jnp.float32) + bps[...]
    hd_lp[...] = jnp.dot(p, wld[...], preferred_element_type=jnp.float32) + bld[...]
    hd_pl[...] = jnp.dot(p, wpd[...], preferred_element_type=jnp.float32) + bpd[...]


def _edge_proj_body(el, ep, wle, wpe, ble, bpe, fl, fp, out_l, out_p):
    wl = wle[...] * fl[...]
    wp = wpe[...] * fp[...]
    out_l[...] = jnp.dot(el[...], wl, preferred_element_type=jnp.float32) + ble[...] * fl[...]
    out_p[...] = jnp.dot(ep[...], wp, preferred_element_type=jnp.float32) + bpe[...] * fp[...]


def _combine_body(plp, ppl, olp, opl):
    a = jnp.sum(plp[...], axis=0)
    b = jnp.sum(ppl[...], axis=0)
    olp[...] = jnp.sum(a, axis=1, keepdims=True)
    opl[...] = jnp.sum(b, axis=1, keepdims=True)


def _sc_affinity(hs_lp, hd_lp, hs_pl, hd_pl, eh_lp, eh_pl,
                 lp_src, lp_dst, pl_src, pl_dst, lp_gid, pl_gid,
                 bfc_lp, bfc_pl,
                 out_lp, out_pl,
                 idxa0, idxa1, idxb0, idxb1, gidb0, gidb1,
                 rowsa0, rowsa1, rowsb0, rowsb1, rowse0, rowse1,
                 acc_lp, acc_pl, bfcbuf,
                 sidx0, sidx1, srows0, srows1):
    cid = lax.axis_index("c")
    sid = lax.axis_index("s")
    wid = sid * _NC + cid

    ntrip = -(-_NCHUNK // _NW)
    assert ntrip % 2 == 0

    idxa = [idxa0, idxa1]
    idxb = [idxb0, idxb1]
    gidb = [gidb0, gidb1]
    rowsa = [rowsa0, rowsa1]
    rowsb = [rowsb0, rowsb1]
    rowse = [rowse0, rowse1]
    sidx = [sidx0, sidx1]
    srows = [srows0, srows1]

    zero16 = jnp.zeros((16,), jnp.float32)
    for i in range(_G):
        acc_lp[i] = zero16
        acc_pl[i] = zero16

    lane = lax.iota(jnp.int32, 16)
    nhblk = 8
    hunroll = _HID // nhblk

    def run_dir(src_hbm, dst_hbm, gid_hbm, taba, tabb, ehtab, acc, bfc_hbm):
        pltpu.sync_copy(bfc_hbm, bfcbuf)
        bfcv = bfcbuf[...]

        def idx_copies(ci, b):
            e0 = ci * _C
            return [
                pltpu.make_async_copy(src_hbm.at[pl.ds(e0, _C)], idxa[b], sidx[b]),
                pltpu.make_async_copy(dst_hbm.at[pl.ds(e0, _C)], idxb[b], sidx[b]),
                pltpu.make_async_copy(gid_hbm.at[pl.ds(e0, _C)], gidb[b], sidx[b]),
            ]

        def rows_copies(ci, b):
            cps = []
            for k in range(_C // 128):
                sl = pl.ds(k * 128, 128)
                cps.append(pltpu.make_async_copy(
                    taba.at[idxa[b].at[sl]], rowsa[b].at[sl], srows[b]))
                cps.append(pltpu.make_async_copy(
                    tabb.at[idxb[b].at[sl]], rowsb[b].at[sl], srows[b]))
            cps.append(pltpu.make_async_copy(
                ehtab.at[pl.ds(ci * _C, _C)], rowse[b], srows[b]))
            return cps

        def issue_idx(j, b):
            ci = wid + _NW * j

            @pl.when((j < ntrip) & (ci < _NCHUNK))
            def _():
                for cp in idx_copies(ci, b):
                    cp.start()

        def start_rows(j, b):
            ci = wid + _NW * j

            @pl.when((j < ntrip) & (ci < _NCHUNK))
            def _():
                for cp in idx_copies(ci, b):
                    cp.wait()
                for cp in rows_copies(ci, b):
                    cp.start()

        def compute(j, b):
            ci = wid + _NW * j

            @pl.when(ci < _NCHUNK)
            def _():
                for cp in rows_copies(ci, b):
                    cp.wait()

                def grp_body(g, _):
                    eidx = lane + g * 16
                    gv = gidb[b][pl.ds(g * 16, 16)]

                    def h_body(hb, v):
                        vv = list(v)
                        base = hb * hunroll
                        for hh in range(hunroll):
                            hv = jnp.zeros((16,), jnp.int32) + (base + hh)
                            a = plsc.load_gather(rowse[b], [eidx, hv])
                            bb = plsc.load_gather(rowsa[b], [eidx, hv])
                            c = plsc.load_gather(rowsb[b], [eidx, hv])
                            vv[hh % 4] = vv[hh % 4] + a * bb * c
                        return tuple(vv)

                    v = lax.fori_loop(0, nhblk, h_body,
                                      (zero16, zero16, zero16, zero16))
                    val = (v[0] + v[1]) + (v[2] + v[3]) + bfcv
                    plsc.addupdate_scatter(acc, [gv, lane], val)
                    return 0

                lax.fori_loop(0, _C // 16, grp_body, 0)

        issue_idx(0, 0)
        start_rows(0, 0)
        issue_idx(1, 1)

        def pair_body(p, _):
            for b in (0, 1):
                j = 2 * p + b
                start_rows(j + 1, 1 - b)
                compute(j, b)
                issue_idx(j + 2, b)
            return 0

        lax.fori_loop(0, ntrip // 2, pair_body, 0)

    run_dir(lp_src, lp_dst, lp_gid, hs_lp, hd_lp, eh_lp, acc_lp, bfc_lp)
    run_dir(pl_src, pl_dst, pl_gid, hd_pl, hs_pl, eh_pl, acc_pl, bfc_pl)

    pltpu.sync_copy(acc_lp, out_lp.at[wid])
    pltpu.sync_copy(acc_pl, out_pl.at[wid])


def kernel(h_ligand, h_pocket, e_lp, e_pl, lp_src, lp_dst, pl_src, pl_dst,
           lp_graph_id, pl_graph_id,
           W_lp_src, b_lp_src, W_lp_dst, b_lp_dst, W_lp_edge, b_lp_edge,
           W_pl_src, b_pl_src, W_pl_dst, b_pl_dst, W_pl_edge, b_pl_edge,
           W_fc_lp, b_fc_lp, W_fc_pl, b_fc_pl):
    f32 = jnp.float32
    i32 = jnp.int32

    def padw(w):
        return jnp.pad(w, ((0, 0), (0, _ROW - _HID)))

    def padb(b):
        return jnp.pad(b.reshape(1, _HID), ((0, 0), (0, _ROW - _HID)))

    n_grid = _N_NODE // _NODE_BLK
    node_spec = pl.BlockSpec((_NODE_BLK, _NODE_F), lambda i: (i, 0))
    nw_spec = pl.BlockSpec((_NODE_F, _ROW), lambda i: (0, 0))
    nb_spec = pl.BlockSpec((1, _ROW), lambda i: (0, 0))
    nout_spec = pl.BlockSpec((_NODE_BLK, _ROW), lambda i: (i, 0))
    hs_lp, hs_pl, hd_lp, hd_pl = pl.pallas_call(
        _node_proj_body,
        grid=(n_grid,),
        in_specs=[node_spec, node_spec, nw_spec, nw_spec, nw_spec, nw_spec,
                  nb_spec, nb_spec, nb_spec, nb_spec],
        out_specs=[nout_spec] * 4,
        out_shape=[jax.ShapeDtypeStruct((_N_NODE, _ROW), f32)] * 4,
    )(h_ligand, h_pocket, padw(W_lp_src), padw(W_pl_src), padw(W_lp_dst),
      padw(W_pl_dst),
      padb(b_lp_src), padb(b_pl_src), padb(b_lp_dst), padb(b_pl_dst))

    e_grid = _E // _EDGE_BLK
    edge_spec = pl.BlockSpec((_EDGE_BLK, _EDGE_F), lambda i: (i, 0))
    ew_spec = pl.BlockSpec((_EDGE_F, _ROW), lambda i: (0, 0))
    eb_spec = pl.BlockSpec((1, _ROW), lambda i: (0, 0))
    eout_spec = pl.BlockSpec((_EDGE_BLK, _ROW), lambda i: (i, 0))
    eh_lp, eh_pl = pl.pallas_call(
        _edge_proj_body,
        grid=(e_grid,),
        in_specs=[edge_spec, edge_spec, ew_spec, ew_spec,
                  eb_spec, eb_spec, eb_spec, eb_spec],
        out_specs=[eout_spec] * 2,
        out_shape=[jax.ShapeDtypeStruct((_E, _ROW), f32)] * 2,
    )(e_lp, e_pl, padw(W_lp_edge), padw(W_pl_edge),
      padb(b_lp_edge), padb(b_pl_edge),
      padw(W_fc_lp.reshape(1, _HID)), padw(W_fc_pl.reshape(1, _HID)))

    mesh = plsc.VectorSubcoreMesh(core_axis_name="c", subcore_axis_name="s",
                                  num_cores=_NC)
    sc_fn = pl.kernel(
        _sc_affinity,
        mesh=mesh,
        compiler_params=pltpu.CompilerParams(
            use_tc_tiling_on_sc=False, needs_layout_passes=False),
        out_type=[jax.ShapeDtypeStruct((_NW, _G, 16), f32)] * 2,
        scratch_types=(
            [pltpu.VMEM((_C,), i32)] * 6
            + [pltpu.VMEM((_C, _ROW), f32)] * 6
            + [pltpu.VMEM((_G, 16), f32)] * 2
            + [pltpu.VMEM((16,), f32)]
            + [pltpu.SemaphoreType.DMA] * 4
        ),
    )
    part_lp, part_pl = sc_fn(
        hs_lp, hd_lp, hs_pl, hd_pl, eh_lp, eh_pl,
        lp_src.astype(i32), lp_dst.astype(i32),
        pl_src.astype(i32), pl_dst.astype(i32),
        lp_graph_id.astype(i32), pl_graph_id.astype(i32),
        jnp.broadcast_to(b_fc_lp.astype(f32), (16,)),
        jnp.broadcast_to(b_fc_pl.astype(f32), (16,)))

    logit_lp, logit_pl = pl.pallas_call(
        _combine_body,
        out_shape=[jax.ShapeDtypeStruct((_G, 1), f32)] * 2,
    )(part_lp, part_pl)
    return (logit_lp, logit_pl)


# speedup vs baseline: 3.8737x; 1.4719x over previous
import functools

import jax
import jax.numpy as jnp
from jax import lax
from jax.experimental import pallas as pl
from jax.experimental.pallas import tpu as pltpu
from jax.experimental.pallas import tpu_sc as plsc

_N_NODE = 50000
_E = 800000
_NODE_F = 128
_EDGE_F = 16
_HID = 64
_G = 64

_C = 256
_NCHUNK = _E // _C
_NC = 2
_NS = 16
_NW = _NC * _NS

_NODE_BLK = 2000
_EDGE_BLK = 8000


def _node_proj_body(hl, hp, wls, wps, wld, wpd, bls, bps, bld, bpd,
                    hs_lp, hs_pl, hd_lp, hd_pl):
    l = hl[...]
    p = hp[...]
    hs_lp[...] = jnp.dot(l, wls[...], preferred_element_type=jnp.float32) + bls[...]
    hs_pl[...] = jnp.dot(l, wps[...], preferred_element_type=jnp.float32) + bps[...]
    hd_lp[...] = jnp.dot(p, wld[...], preferred_element_type=jnp.float32) + bld[...]
    hd_pl[...] = jnp.dot(p, wpd[...], preferred_element_type=jnp.float32) + bpd[...]


def _edge_proj_body(el, ep, wle, wpe, ble, bpe, fl, fp, out_l, out_p):
    wl = wle[...] * fl[...]
    wp = wpe[...] * fp[...]
    out_l[...] = jnp.dot(el[...], wl, preferred_element_type=jnp.float32) + ble[...] * fl[...]
    out_p[...] = jnp.dot(ep[...], wp, preferred_element_type=jnp.float32) + bpe[...] * fp[...]


def _combine_body(plp, ppl, olp, opl):
    a = jnp.sum(plp[...], axis=0)
    b = jnp.sum(ppl[...], axis=0)
    olp[...] = jnp.sum(a, axis=1, keepdims=True)
    opl[...] = jnp.sum(b, axis=1, keepdims=True)


def _sc_affinity(hs_lp, hd_lp, hs_pl, hd_pl, eh_lp, eh_pl,
                 lp_src, lp_dst, pl_src, pl_dst, lp_gid, pl_gid,
                 bfc_lp, bfc_pl,
                 out_lp, out_pl,
                 idxa0, idxa1, idxb0, idxb1, gidb0, gidb1,
                 rowsa0, rowsa1, rowsb0, rowsb1, rowse0, rowse1,
                 acc_lp, acc_pl, bfcbuf,
                 sidx0, sidx1, srows0, srows1):
    cid = lax.axis_index("c")
    sid = lax.axis_index("s")
    wid = sid * _NC + cid

    ntrip = -(-_NCHUNK // _NW)
    assert ntrip % 2 == 0

    idxa = [idxa0, idxa1]
    idxb = [idxb0, idxb1]
    gidb = [gidb0, gidb1]
    rowsa = [rowsa0, rowsa1]
    rowsb = [rowsb0, rowsb1]
    rowse = [rowse0, rowse1]
    sidx = [sidx0, sidx1]
    srows = [srows0, srows1]

    zero16 = jnp.zeros((16,), jnp.float32)
    for i in range(_G):
        acc_lp[i] = zero16
        acc_pl[i] = zero16

    lane = lax.iota(jnp.int32, 16)
    nhblk = 8
    hunroll = _HID // nhblk

    def run_dir(src_hbm, dst_hbm, gid_hbm, taba, tabb, ehtab, acc, bfc_hbm):
        pltpu.sync_copy(bfc_hbm, bfcbuf)
        bfcv = bfcbuf[...]

        def idx_copies(ci, b):
            e0 = ci * _C
            return [
                pltpu.make_async_copy(src_hbm.at[pl.ds(e0, _C)], idxa[b], sidx[b]),
                pltpu.make_async_copy(dst_hbm.at[pl.ds(e0, _C)], idxb[b], sidx[b]),
                pltpu.make_async_copy(gid_hbm.at[pl.ds(e0, _C)], gidb[b], sidx[b]),
            ]

        def rows_copies(ci, b):
            cps = []
            for k in range(_C // 128):
                sl = pl.ds(k * 128, 128)
                cps.append(pltpu.make_async_copy(
                    taba.at[idxa[b].at[sl]], rowsa[b].at[sl], srows[b]))
                cps.append(pltpu.make_async_copy(
                    tabb.at[idxb[b].at[sl]], rowsb[b].at[sl], srows[b]))
            cps.append(pltpu.make_async_copy(
                ehtab.at[pl.ds(ci * _C, _C)], rowse[b], srows[b]))
            return cps

        def issue_idx(j, b):
            ci = wid + _NW * j

            @pl.when((j < ntrip) & (ci < _NCHUNK))
            def _():
                for cp in idx_copies(ci, b):
                    cp.start()

        def start_rows(j, b):
            ci = wid + _NW * j

            @pl.when((j < ntrip) & (ci < _NCHUNK))
            def _():
                for cp in idx_copies(ci, b):
                    cp.wait()
                for cp in rows_copies(ci, b):
                    cp.start()

        def compute(j, b):
            ci = wid + _NW * j

            @pl.when(ci < _NCHUNK)
            def _():
                for cp in rows_copies(ci, b):
                    cp.wait()

                def grp_body(g, _):
                    eidx = lane + g * 16
                    gv = gidb[b][pl.ds(g * 16, 16)]

                    def h_body(hb, v):
                        vv = list(v)
                        base = hb * hunroll
                        for hh in range(hunroll):
                            hv = jnp.bitwise_and(lane + (base + hh), _HID - 1)
                            a = plsc.load_gather(rowse[b], [eidx, hv])
                            bb = plsc.load_gather(rowsa[b], [eidx, hv])
                            c = plsc.load_gather(rowsb[b], [eidx, hv])
                            vv[hh % 4] = vv[hh % 4] + a * bb * c
                        return tuple(vv)

                    v = lax.fori_loop(0, nhblk, h_body,
                                      (zero16, zero16, zero16, zero16))
                    val = (v[0] + v[1]) + (v[2] + v[3]) + bfcv
                    plsc.addupdate_scatter(acc, [gv, lane], val)
                    return 0

                lax.fori_loop(0, _C // 16, grp_body, 0)

        issue_idx(0, 0)
        start_rows(0, 0)
        issue_idx(1, 1)

        def pair_body(p, _):
            for b in (0, 1):
                j = 2 * p + b
                start_rows(j + 1, 1 - b)
                compute(j, b)
                issue_idx(j + 2, b)
            return 0

        lax.fori_loop(0, ntrip // 2, pair_body, 0)

    run_dir(lp_src, lp_dst, lp_gid, hs_lp, hd_lp, eh_lp, acc_lp, bfc_lp)
    run_dir(pl_src, pl_dst, pl_gid, hd_pl, hs_pl, eh_pl, acc_pl, bfc_pl)

    pltpu.sync_copy(acc_lp, out_lp.at[wid])
    pltpu.sync_copy(acc_pl, out_pl.at[wid])


def kernel(h_ligand, h_pocket, e_lp, e_pl, lp_src, lp_dst, pl_src, pl_dst,
           lp_graph_id, pl_graph_id,
           W_lp_src, b_lp_src, W_lp_dst, b_lp_dst, W_lp_edge, b_lp_edge,
           W_pl_src, b_pl_src, W_pl_dst, b_pl_dst, W_pl_edge, b_pl_edge,
           W_fc_lp, b_fc_lp, W_fc_pl, b_fc_pl):
    f32 = jnp.float32
    i32 = jnp.int32

    n_grid = _N_NODE // _NODE_BLK
    node_spec = pl.BlockSpec((_NODE_BLK, _NODE_F), lambda i: (i, 0))
    nw_spec = pl.BlockSpec((_NODE_F, _HID), lambda i: (0, 0))
    nb_spec = pl.BlockSpec((1, _HID), lambda i: (0, 0))
    nout_spec = pl.BlockSpec((_NODE_BLK, _HID), lambda i: (i, 0))
    hs_lp, hs_pl, hd_lp, hd_pl = pl.pallas_call(
        _node_proj_body,
        grid=(n_grid,),
        in_specs=[node_spec, node_spec, nw_spec, nw_spec, nw_spec, nw_spec,
                  nb_spec, nb_spec, nb_spec, nb_spec],
        out_specs=[nout_spec] * 4,
        out_shape=[jax.ShapeDtypeStruct((_N_NODE, _HID), f32)] * 4,
    )(h_ligand, h_pocket, W_lp_src, W_pl_src, W_lp_dst, W_pl_dst,
      b_lp_src.reshape(1, _HID), b_pl_src.reshape(1, _HID),
      b_lp_dst.reshape(1, _HID), b_pl_dst.reshape(1, _HID))

    e_grid = _E // _EDGE_BLK
    edge_spec = pl.BlockSpec((_EDGE_BLK, _EDGE_F), lambda i: (i, 0))
    ew_spec = pl.BlockSpec((_EDGE_F, _HID), lambda i: (0, 0))
    eb_spec = pl.BlockSpec((1, _HID), lambda i: (0, 0))
    eout_spec = pl.BlockSpec((_EDGE_BLK, _HID), lambda i: (i, 0))
    eh_lp, eh_pl = pl.pallas_call(
        _edge_proj_body,
        grid=(e_grid,),
        in_specs=[edge_spec, edge_spec, ew_spec, ew_spec,
                  eb_spec, eb_spec, eb_spec, eb_spec],
        out_specs=[eout_spec] * 2,
        out_shape=[jax.ShapeDtypeStruct((_E, _HID), f32)] * 2,
    )(e_lp, e_pl, W_lp_edge, W_pl_edge,
      b_lp_edge.reshape(1, _HID), b_pl_edge.reshape(1, _HID),
      W_fc_lp.reshape(1, _HID), W_fc_pl.reshape(1, _HID))

    mesh = plsc.VectorSubcoreMesh(core_axis_name="c", subcore_axis_name="s",
                                  num_cores=_NC)
    sc_fn = pl.kernel(
        _sc_affinity,
        mesh=mesh,
        compiler_params=pltpu.CompilerParams(
            use_tc_tiling_on_sc=False, needs_layout_passes=False),
        out_type=[jax.ShapeDtypeStruct((_NW, _G, 16), f32)] * 2,
        scratch_types=(
            [pltpu.VMEM((_C,), i32)] * 6
            + [pltpu.VMEM((_C, _HID), f32)] * 6
            + [pltpu.VMEM((_G, 16), f32)] * 2
            + [pltpu.VMEM((16,), f32)]
            + [pltpu.SemaphoreType.DMA] * 4
        ),
    )
    part_lp, part_pl = sc_fn(
        hs_lp, hd_lp, hs_pl, hd_pl, eh_lp, eh_pl,
        lp_src.astype(i32), lp_dst.astype(i32),
        pl_src.astype(i32), pl_dst.astype(i32),
        lp_graph_id.astype(i32), pl_graph_id.astype(i32),
        jnp.broadcast_to(b_fc_lp.astype(f32), (16,)),
        jnp.broadcast_to(b_fc_pl.astype(f32), (16,)))

    logit_lp, logit_pl = pl.pallas_call(
        _combine_body,
        out_shape=[jax.ShapeDtypeStruct((_G, 1), f32)] * 2,
    )(part_lp, part_pl)
    return (logit_lp, logit_pl)


# speedup vs baseline: 4.0930x; 1.0566x over previous
import functools

import jax
import jax.numpy as jnp
from jax import lax
from jax.experimental import pallas as pl
from jax.experimental.pallas import tpu as pltpu
from jax.experimental.pallas import tpu_sc as plsc

_N_NODE = 50000
_E = 800000
_NODE_F = 128
_EDGE_F = 16
_HID = 64
_G = 64

_C = 256
_NCHUNK = _E // _C
_NC = 2
_NS = 16
_NW = _NC * _NS

_NODE_BLK = 2000
_EDGE_BLK = 8000


def _node_proj_body(hl, hp, wls, wps, wld, wpd, bls, bps, bld, bpd,
                    hs_lp, hs_pl, hd_lp, hd_pl):
    l = hl[...]
    p = hp[...]
    hs_lp[...] = jnp.dot(l, wls[...], preferred_element_type=jnp.float32) + bls[...]
    hs_pl[...] = jnp.dot(l, wps[...], preferred_element_type=jnp.float32) + bps[...]
    hd_lp[...] = jnp.dot(p, wld[...], preferred_element_type=jnp.float32) + bld[...]
    hd_pl[...] = jnp.dot(p, wpd[...], preferred_element_type=jnp.float32) + bpd[...]


def _edge_proj_body(el, wle, ble, fl, out_l):
    wl = wle[...] * fl[...]
    out_l[...] = jnp.dot(el[...], wl, preferred_element_type=jnp.float32) + ble[...] * fl[...]


def _combine_body(plp, ppl, olp, opl):
    a = jnp.sum(plp[...], axis=0)
    b = jnp.sum(ppl[...], axis=0)
    olp[...] = jnp.sum(a, axis=1, keepdims=True)
    opl[...] = jnp.sum(b, axis=1, keepdims=True)


def _sc_dir(taba, tabb, ehtab, src_hbm, dst_hbm, gid_hbm, bfc_hbm,
            out_part,
            idxa0, idxa1, idxb0, idxb1, gidb0, gidb1,
            rowsa0, rowsa1, rowsb0, rowsb1, rowse0, rowse1,
            acc, bfcbuf,
            sidx0, sidx1, srows0, srows1):
    cid = lax.axis_index("c")
    sid = lax.axis_index("s")
    wid = sid * _NC + cid

    ntrip = -(-_NCHUNK // _NW)
    assert ntrip % 2 == 0

    idxa = [idxa0, idxa1]
    idxb = [idxb0, idxb1]
    gidb = [gidb0, gidb1]
    rowsa = [rowsa0, rowsa1]
    rowsb = [rowsb0, rowsb1]
    rowse = [rowse0, rowse1]
    sidx = [sidx0, sidx1]
    srows = [srows0, srows1]

    zero16 = jnp.zeros((16,), jnp.float32)
    for i in range(_G):
        acc[i] = zero16

    lane = lax.iota(jnp.int32, 16)
    nhblk = 8
    hunroll = _HID // nhblk

    pltpu.sync_copy(bfc_hbm, bfcbuf)
    bfcv = bfcbuf[...]

    def idx_copies(ci, b):
        e0 = ci * _C
        return [
            pltpu.make_async_copy(src_hbm.at[pl.ds(e0, _C)], idxa[b], sidx[b]),
            pltpu.make_async_copy(dst_hbm.at[pl.ds(e0, _C)], idxb[b], sidx[b]),
            pltpu.make_async_copy(gid_hbm.at[pl.ds(e0, _C)], gidb[b], sidx[b]),
        ]

    def rows_copies(ci, b):
        cps = []
        for k in range(_C // 128):
            sl = pl.ds(k * 128, 128)
            cps.append(pltpu.make_async_copy(
                taba.at[idxa[b].at[sl]], rowsa[b].at[sl], srows[b]))
            cps.append(pltpu.make_async_copy(
                tabb.at[idxb[b].at[sl]], rowsb[b].at[sl], srows[b]))
        cps.append(pltpu.make_async_copy(
            ehtab.at[pl.ds(ci * _C, _C)], rowse[b], srows[b]))
        return cps

    def issue_idx(j, b):
        ci = wid + _NW * j

        @pl.when((j < ntrip) & (ci < _NCHUNK))
        def _():
            for cp in idx_copies(ci, b):
                cp.start()

    def start_rows(j, b):
        ci = wid + _NW * j

        @pl.when((j < ntrip) & (ci < _NCHUNK))
        def _():
            for cp in idx_copies(ci, b):
                cp.wait()
            for cp in rows_copies(ci, b):
                cp.start()

    def compute(j, b):
        ci = wid + _NW * j

        @pl.when(ci < _NCHUNK)
        def _():
            for cp in rows_copies(ci, b):
                cp.wait()

            def grp_body(g, _):
                eidx = lane + g * 16
                gv = gidb[b][pl.ds(g * 16, 16)]

                def h_body(hb, v):
                    vv = list(v)
                    base = hb * hunroll
                    for hh in range(hunroll):
                        hv = jnp.bitwise_and(lane + (base + hh), _HID - 1)
                        a = plsc.load_gather(rowse[b], [eidx, hv])
                        bb = plsc.load_gather(rowsa[b], [eidx, hv])
                        c = plsc.load_gather(rowsb[b], [eidx, hv])
                        vv[hh % 4] = vv[hh % 4] + a * bb * c
                    return tuple(vv)

                v = lax.fori_loop(0, nhblk, h_body,
                                  (zero16, zero16, zero16, zero16))
                val = (v[0] + v[1]) + (v[2] + v[3]) + bfcv
                plsc.addupdate_scatter(acc, [gv, lane], val)
                return 0

            lax.fori_loop(0, _C // 16, grp_body, 0)

    issue_idx(0, 0)
    start_rows(0, 0)
    issue_idx(1, 1)

    def pair_body(p, _):
        for b in (0, 1):
            j = 2 * p + b
            start_rows(j + 1, 1 - b)
            compute(j, b)
            issue_idx(j + 2, b)
        return 0

    lax.fori_loop(0, ntrip // 2, pair_body, 0)

    pltpu.sync_copy(acc, out_part.at[wid])


def kernel(h_ligand, h_pocket, e_lp, e_pl, lp_src, lp_dst, pl_src, pl_dst,
           lp_graph_id, pl_graph_id,
           W_lp_src, b_lp_src, W_lp_dst, b_lp_dst, W_lp_edge, b_lp_edge,
           W_pl_src, b_pl_src, W_pl_dst, b_pl_dst, W_pl_edge, b_pl_edge,
           W_fc_lp, b_fc_lp, W_fc_pl, b_fc_pl):
    f32 = jnp.float32
    i32 = jnp.int32

    n_grid = _N_NODE // _NODE_BLK
    node_spec = pl.BlockSpec((_NODE_BLK, _NODE_F), lambda i: (i, 0))
    nw_spec = pl.BlockSpec((_NODE_F, _HID), lambda i: (0, 0))
    nb_spec = pl.BlockSpec((1, _HID), lambda i: (0, 0))
    nout_spec = pl.BlockSpec((_NODE_BLK, _HID), lambda i: (i, 0))
    hs_lp, hs_pl, hd_lp, hd_pl = pl.pallas_call(
        _node_proj_body,
        grid=(n_grid,),
        in_specs=[node_spec, node_spec, nw_spec, nw_spec, nw_spec, nw_spec,
                  nb_spec, nb_spec, nb_spec, nb_spec],
        out_specs=[nout_spec] * 4,
        out_shape=[jax.ShapeDtypeStruct((_N_NODE, _HID), f32)] * 4,
    )(h_ligand, h_pocket, W_lp_src, W_pl_src, W_lp_dst, W_pl_dst,
      b_lp_src.reshape(1, _HID), b_pl_src.reshape(1, _HID),
      b_lp_dst.reshape(1, _HID), b_pl_dst.reshape(1, _HID))

    e_grid = _E // _EDGE_BLK
    edge_spec = pl.BlockSpec((_EDGE_BLK, _EDGE_F), lambda i: (i, 0))
    ew_spec = pl.BlockSpec((_EDGE_F, _HID), lambda i: (0, 0))
    eb_spec = pl.BlockSpec((1, _HID), lambda i: (0, 0))
    eout_spec = pl.BlockSpec((_EDGE_BLK, _HID), lambda i: (i, 0))
    edge_proj = pl.pallas_call(
        _edge_proj_body,
        grid=(e_grid,),
        in_specs=[edge_spec, ew_spec, eb_spec, eb_spec],
        out_specs=[eout_spec],
        out_shape=[jax.ShapeDtypeStruct((_E, _HID), f32)],
    )

    mesh = plsc.VectorSubcoreMesh(core_axis_name="c", subcore_axis_name="s",
                                  num_cores=_NC)
    sc_dir = pl.kernel(
        _sc_dir,
        mesh=mesh,
        compiler_params=pltpu.CompilerParams(
            use_tc_tiling_on_sc=False, needs_layout_passes=False),
        out_type=jax.ShapeDtypeStruct((_NW, _G, 16), f32),
        scratch_types=(
            [pltpu.VMEM((_C,), i32)] * 6
            + [pltpu.VMEM((_C, _HID), f32)] * 6
            + [pltpu.VMEM((_G, 16), f32)]
            + [pltpu.VMEM((16,), f32)]
            + [pltpu.SemaphoreType.DMA] * 4
        ),
    )

    (eh_lp,) = edge_proj(e_lp, W_lp_edge, b_lp_edge.reshape(1, _HID),
                         W_fc_lp.reshape(1, _HID))
    part_lp = sc_dir(
        hs_lp, hd_lp, eh_lp,
        lp_src.astype(i32), lp_dst.astype(i32), lp_graph_id.astype(i32),
        jnp.broadcast_to(b_fc_lp.astype(f32), (16,)))

    (eh_pl,) = edge_proj(e_pl, W_pl_edge, b_pl_edge.reshape(1, _HID),
                         W_fc_pl.reshape(1, _HID))
    part_pl = sc_dir(
        hd_pl, hs_pl, eh_pl,
        pl_src.astype(i32), pl_dst.astype(i32), pl_graph_id.astype(i32),
        jnp.broadcast_to(b_fc_pl.astype(f32), (16,)))

    logit_lp, logit_pl = pl.pallas_call(
        _combine_body,
        out_shape=[jax.ShapeDtypeStruct((_G, 1), f32)] * 2,
    )(part_lp, part_pl)
    return (logit_lp, logit_pl)


# speedup vs baseline: 4.5328x; 1.1074x over previous
import functools

import jax
import jax.numpy as jnp
from jax import lax
from jax.experimental import pallas as pl
from jax.experimental.pallas import tpu as pltpu
from jax.experimental.pallas import tpu_sc as plsc

_N_NODE = 50000
_E = 800000
_NODE_F = 128
_EDGE_F = 16
_HID = 64
_G = 64

_C = 256
_NCHUNK = _E // _C
_NC = 2
_NS = 16
_NW = _NC * _NS

_NODE_BLK = 2000
_EDGE_BLK = 8000


def _pack_pairs(y):
    lo = jax.lax.bitcast_convert_type(
        y[:, :_HID // 2].astype(jnp.bfloat16), jnp.uint16).astype(jnp.uint32)
    hi = jax.lax.bitcast_convert_type(
        y[:, _HID // 2:].astype(jnp.bfloat16), jnp.uint16).astype(jnp.uint32)
    return jax.lax.bitcast_convert_type(lo | (hi << 16), jnp.int32)


def _node_proj_body(hl, hp, wls, wps, wld, wpd, bls, bps, bld, bpd,
                    hs_lp, hs_pl, hd_lp, hd_pl):
    l = hl[...]
    p = hp[...]
    hs_lp[...] = _pack_pairs(jnp.dot(l, wls[...], preferred_element_type=jnp.float32) + bls[...])
    hs_pl[...] = _pack_pairs(jnp.dot(l, wps[...], preferred_element_type=jnp.float32) + bps[...])
    hd_lp[...] = _pack_pairs(jnp.dot(p, wld[...], preferred_element_type=jnp.float32) + bld[...])
    hd_pl[...] = _pack_pairs(jnp.dot(p, wpd[...], preferred_element_type=jnp.float32) + bpd[...])


def _edge_proj_body(el, wle, ble, fl, out_l):
    wl = wle[...] * fl[...]
    out_l[...] = _pack_pairs(
        jnp.dot(el[...], wl, preferred_element_type=jnp.float32) + ble[...] * fl[...])


def _combine_body(plp, ppl, olp, opl):
    a = jnp.sum(plp[...], axis=0)
    b = jnp.sum(ppl[...], axis=0)
    olp[...] = jnp.sum(a, axis=1, keepdims=True)
    opl[...] = jnp.sum(b, axis=1, keepdims=True)


def _sc_dir(taba, tabb, ehtab, src_hbm, dst_hbm, gid_hbm, bfc_hbm,
            out_part,
            idxa0, idxa1, idxb0, idxb1, gidb0, gidb1,
            rowsa0, rowsa1, rowsb0, rowsb1, rowse0, rowse1,
            acc, bfcbuf,
            sidx0, sidx1, srows0, srows1):
    cid = lax.axis_index("c")
    sid = lax.axis_index("s")
    wid = sid * _NC + cid

    ntrip = -(-_NCHUNK // _NW)
    assert ntrip % 2 == 0

    idxa = [idxa0, idxa1]
    idxb = [idxb0, idxb1]
    gidb = [gidb0, gidb1]
    rowsa = [rowsa0, rowsa1]
    rowsb = [rowsb0, rowsb1]
    rowse = [rowse0, rowse1]
    sidx = [sidx0, sidx1]
    srows = [srows0, srows1]

    zero16 = jnp.zeros((16,), jnp.float32)
    for i in range(_G):
        acc[i] = zero16

    lane = lax.iota(jnp.int32, 16)
    nw_words = _HID // 2
    nwblk = 4
    wunroll = nw_words // nwblk

    def unlo(x):
        return plsc.bitcast(jax.lax.shift_left(x, 16), jnp.float32)

    def unhi(x):
        return plsc.bitcast(jnp.bitwise_and(x, jnp.int32(-65536)), jnp.float32)

    pltpu.sync_copy(bfc_hbm, bfcbuf)
    bfcv = bfcbuf[...]

    def idx_copies(ci, b):
        e0 = ci * _C
        return [
            pltpu.make_async_copy(src_hbm.at[pl.ds(e0, _C)], idxa[b], sidx[b]),
            pltpu.make_async_copy(dst_hbm.at[pl.ds(e0, _C)], idxb[b], sidx[b]),
            pltpu.make_async_copy(gid_hbm.at[pl.ds(e0, _C)], gidb[b], sidx[b]),
        ]

    def rows_copies(ci, b):
        cps = []
        for k in range(_C // 128):
            sl = pl.ds(k * 128, 128)
            cps.append(pltpu.make_async_copy(
                taba.at[idxa[b].at[sl]], rowsa[b].at[sl], srows[b]))
            cps.append(pltpu.make_async_copy(
                tabb.at[idxb[b].at[sl]], rowsb[b].at[sl], srows[b]))
        cps.append(pltpu.make_async_copy(
            ehtab.at[pl.ds(ci * _C, _C)], rowse[b], srows[b]))
        return cps

    def issue_idx(j, b):
        ci = wid + _NW * j

        @pl.when((j < ntrip) & (ci < _NCHUNK))
        def _():
            for cp in idx_copies(ci, b):
                cp.start()

    def start_rows(j, b):
        ci = wid + _NW * j

        @pl.when((j < ntrip) & (ci < _NCHUNK))
        def _():
            for cp in idx_copies(ci, b):
                cp.wait()
            for cp in rows_copies(ci, b):
                cp.start()

    def compute(j, b):
        ci = wid + _NW * j

        @pl.when(ci < _NCHUNK)
        def _():
            for cp in rows_copies(ci, b):
                cp.wait()

            def grp_body(g, _):
                eidx = lane + g * 16
                gv = gidb[b][pl.ds(g * 16, 16)]

                def h_body(hb, v):
                    vv = list(v)
                    base = hb * wunroll
                    for ww in range(wunroll):
                        wv = jnp.bitwise_and(lane + (base + ww), nw_words - 1)
                        a = plsc.load_gather(rowse[b], [eidx, wv])
                        bb = plsc.load_gather(rowsa[b], [eidx, wv])
                        c = plsc.load_gather(rowsb[b], [eidx, wv])
                        vv[(2 * ww) % 4] = (vv[(2 * ww) % 4]
                                            + unlo(a) * unlo(bb) * unlo(c))
                        vv[(2 * ww + 1) % 4] = (vv[(2 * ww + 1) % 4]
                                                + unhi(a) * unhi(bb) * unhi(c))
                    return tuple(vv)

                v = lax.fori_loop(0, nwblk, h_body,
                                  (zero16, zero16, zero16, zero16))
                val = (v[0] + v[1]) + (v[2] + v[3]) + bfcv
                plsc.addupdate_scatter(acc, [gv, lane], val)
                return 0

            lax.fori_loop(0, _C // 16, grp_body, 0)

    issue_idx(0, 0)
    start_rows(0, 0)
    issue_idx(1, 1)

    def pair_body(p, _):
        for b in (0, 1):
            j = 2 * p + b
            start_rows(j + 1, 1 - b)
            compute(j, b)
            issue_idx(j + 2, b)
        return 0

    lax.fori_loop(0, ntrip // 2, pair_body, 0)

    pltpu.sync_copy(acc, out_part.at[wid])


def kernel(h_ligand, h_pocket, e_lp, e_pl, lp_src, lp_dst, pl_src, pl_dst,
           lp_graph_id, pl_graph_id,
           W_lp_src, b_lp_src, W_lp_dst, b_lp_dst, W_lp_edge, b_lp_edge,
           W_pl_src, b_pl_src, W_pl_dst, b_pl_dst, W_pl_edge, b_pl_edge,
           W_fc_lp, b_fc_lp, W_fc_pl, b_fc_pl):
    f32 = jnp.float32
    i32 = jnp.int32

    n_grid = _N_NODE // _NODE_BLK
    node_spec = pl.BlockSpec((_NODE_BLK, _NODE_F), lambda i: (i, 0))
    nw_spec = pl.BlockSpec((_NODE_F, _HID), lambda i: (0, 0))
    nb_spec = pl.BlockSpec((1, _HID), lambda i: (0, 0))
    nout_spec = pl.BlockSpec((_NODE_BLK, _HID // 2), lambda i: (i, 0))
    hs_lp, hs_pl, hd_lp, hd_pl = pl.pallas_call(
        _node_proj_body,
        grid=(n_grid,),
        in_specs=[node_spec, node_spec, nw_spec, nw_spec, nw_spec, nw_spec,
                  nb_spec, nb_spec, nb_spec, nb_spec],
        out_specs=[nout_spec] * 4,
        out_shape=[jax.ShapeDtypeStruct((_N_NODE, _HID // 2), i32)] * 4,
    )(h_ligand, h_pocket, W_lp_src, W_pl_src, W_lp_dst, W_pl_dst,
      b_lp_src.reshape(1, _HID), b_pl_src.reshape(1, _HID),
      b_lp_dst.reshape(1, _HID), b_pl_dst.reshape(1, _HID))

    e_grid = _E // _EDGE_BLK
    edge_spec = pl.BlockSpec((_EDGE_BLK, _EDGE_F), lambda i: (i, 0))
    ew_spec = pl.BlockSpec((_EDGE_F, _HID), lambda i: (0, 0))
    eb_spec = pl.BlockSpec((1, _HID), lambda i: (0, 0))
    eout_spec = pl.BlockSpec((_EDGE_BLK, _HID // 2), lambda i: (i, 0))
    edge_proj = pl.pallas_call(
        _edge_proj_body,
        grid=(e_grid,),
        in_specs=[edge_spec, ew_spec, eb_spec, eb_spec],
        out_specs=[eout_spec],
        out_shape=[jax.ShapeDtypeStruct((_E, _HID // 2), i32)],
    )

    mesh = plsc.VectorSubcoreMesh(core_axis_name="c", subcore_axis_name="s",
                                  num_cores=_NC)
    sc_dir = pl.kernel(
        _sc_dir,
        mesh=mesh,
        compiler_params=pltpu.CompilerParams(
            use_tc_tiling_on_sc=False, needs_layout_passes=False),
        out_type=jax.ShapeDtypeStruct((_NW, _G, 16), f32),
        scratch_types=(
            [pltpu.VMEM((_C,), i32)] * 6
            + [pltpu.VMEM((_C, _HID // 2), i32)] * 6
            + [pltpu.VMEM((_G, 16), f32)]
            + [pltpu.VMEM((16,), f32)]
            + [pltpu.SemaphoreType.DMA] * 4
        ),
    )

    (eh_lp,) = edge_proj(e_lp, W_lp_edge, b_lp_edge.reshape(1, _HID),
                         W_fc_lp.reshape(1, _HID))
    part_lp = sc_dir(
        hs_lp, hd_lp, eh_lp,
        lp_src.astype(i32), lp_dst.astype(i32), lp_graph_id.astype(i32),
        jnp.broadcast_to(b_fc_lp.astype(f32), (16,)))

    (eh_pl,) = edge_proj(e_pl, W_pl_edge, b_pl_edge.reshape(1, _HID),
                         W_fc_pl.reshape(1, _HID))
    part_pl = sc_dir(
        hd_pl, hs_pl, eh_pl,
        pl_src.astype(i32), pl_dst.astype(i32), pl_graph_id.astype(i32),
        jnp.broadcast_to(b_fc_pl.astype(f32), (16,)))

    logit_lp, logit_pl = pl.pallas_call(
        _combine_body,
        out_shape=[jax.ShapeDtypeStruct((_G, 1), f32)] * 2,
    )(part_lp, part_pl)
    return (logit_lp, logit_pl)


# speedup vs baseline: 5.4017x; 1.1917x over previous
import functools

import jax
import jax.numpy as jnp
from jax import lax
from jax.experimental import pallas as pl
from jax.experimental.pallas import tpu as pltpu
from jax.experimental.pallas import tpu_sc as plsc

_N_NODE = 50000
_E = 800000
_NODE_F = 128
_EDGE_F = 16
_HID = 64
_G = 64

_C = 256
_NCHUNK = _E // _C
_NC = 2
_NS = 16
_NW = _NC * _NS

_NODE_BLK = 2000
_EDGE_BLK = 8000


def _pack_pairs(y):
    lo = jax.lax.bitcast_convert_type(
        y[:, :_HID // 2].astype(jnp.bfloat16), jnp.uint16).astype(jnp.uint32)
    hi = jax.lax.bitcast_convert_type(
        y[:, _HID // 2:].astype(jnp.bfloat16), jnp.uint16).astype(jnp.uint32)
    return jax.lax.bitcast_convert_type(lo | (hi << 16), jnp.int32)


def _node_proj_body(hl, hp, wls, wps, wld, wpd, bls, bps, bld, bpd,
                    hs_lp, hs_pl, hd_lp, hd_pl):
    l = hl[...]
    p = hp[...]
    hs_lp[...] = _pack_pairs(jnp.dot(l, wls[...], preferred_element_type=jnp.float32) + bls[...])
    hs_pl[...] = _pack_pairs(jnp.dot(l, wps[...], preferred_element_type=jnp.float32) + bps[...])
    hd_lp[...] = _pack_pairs(jnp.dot(p, wld[...], preferred_element_type=jnp.float32) + bld[...])
    hd_pl[...] = _pack_pairs(jnp.dot(p, wpd[...], preferred_element_type=jnp.float32) + bpd[...])


def _edge_proj_body(er, wb, bbt, out_l):
    y = jnp.dot(er[...], wb[...], preferred_element_type=jnp.float32) + bbt[...]
    for j in range(8):
        out_l[:, j * 32:(j + 1) * 32] = _pack_pairs(y[:, j * 64:(j + 1) * 64])


def _combine_body(plp, ppl, olp, opl):
    a = jnp.sum(plp[...], axis=0)
    b = jnp.sum(ppl[...], axis=0)
    olp[...] = jnp.sum(a, axis=1, keepdims=True)
    opl[...] = jnp.sum(b, axis=1, keepdims=True)


def _sc_dir(taba, tabb, ehtab, src_hbm, dst_hbm, gid_hbm, bfc_hbm,
            out_part,
            idxa0, idxa1, idxb0, idxb1, gidb0, gidb1,
            rowsa0, rowsa1, rowsb0, rowsb1, rowse0, rowse1,
            acc, bfcbuf,
            sidx0, sidx1, srows0, srows1):
    cid = lax.axis_index("c")
    sid = lax.axis_index("s")
    wid = sid * _NC + cid

    ntrip = -(-_NCHUNK // _NW)
    assert ntrip % 2 == 0

    idxa = [idxa0, idxa1]
    idxb = [idxb0, idxb1]
    gidb = [gidb0, gidb1]
    rowsa = [rowsa0, rowsa1]
    rowsb = [rowsb0, rowsb1]
    rowse = [rowse0, rowse1]
    sidx = [sidx0, sidx1]
    srows = [srows0, srows1]

    zero16 = jnp.zeros((16,), jnp.float32)
    for i in range(_G):
        acc[i] = zero16

    lane = lax.iota(jnp.int32, 16)
    nw_words = _HID // 2
    nwblk = 4
    wunroll = nw_words // nwblk

    def unlo(x):
        return plsc.bitcast(jax.lax.shift_left(x, 16), jnp.float32)

    def unhi(x):
        return plsc.bitcast(jnp.bitwise_and(x, jnp.int32(-65536)), jnp.float32)

    pltpu.sync_copy(bfc_hbm, bfcbuf)
    bfcv = bfcbuf[...]

    def idx_copies(ci, b):
        e0 = ci * _C
        return [
            pltpu.make_async_copy(src_hbm.at[pl.ds(e0, _C)], idxa[b], sidx[b]),
            pltpu.make_async_copy(dst_hbm.at[pl.ds(e0, _C)], idxb[b], sidx[b]),
            pltpu.make_async_copy(gid_hbm.at[pl.ds(e0, _C)], gidb[b], sidx[b]),
        ]

    def rows_copies(ci, b):
        cps = []
        for k in range(_C // 128):
            sl = pl.ds(k * 128, 128)
            cps.append(pltpu.make_async_copy(
                taba.at[idxa[b].at[sl]], rowsa[b].at[sl], srows[b]))
            cps.append(pltpu.make_async_copy(
                tabb.at[idxb[b].at[sl]], rowsb[b].at[sl], srows[b]))
        cps.append(pltpu.make_async_copy(
            ehtab.at[pl.ds(ci * (_C // 8), _C // 8)], rowse[b], srows[b]))
        return cps

    def issue_idx(j, b):
        ci = wid + _NW * j

        @pl.when((j < ntrip) & (ci < _NCHUNK))
        def _():
            for cp in idx_copies(ci, b):
                cp.start()

    def start_rows(j, b):
        ci = wid + _NW * j

        @pl.when((j < ntrip) & (ci < _NCHUNK))
        def _():
            for cp in idx_copies(ci, b):
                cp.wait()
            for cp in rows_copies(ci, b):
                cp.start()

    def compute(j, b):
        ci = wid + _NW * j

        @pl.when(ci < _NCHUNK)
        def _():
            for cp in rows_copies(ci, b):
                cp.wait()

            def grp_body(g, _):
                eidx = lane + g * 16
                erow = lax.shift_right_logical(eidx, 3)
                ecol = lax.shift_left(jnp.bitwise_and(eidx, 7), 5)
                gv = gidb[b][pl.ds(g * 16, 16)]

                def h_body(hb, v):
                    vv = list(v)
                    base = hb * wunroll
                    for ww in range(wunroll):
                        wv = jnp.bitwise_and(lane + (base + ww), nw_words - 1)
                        a = plsc.load_gather(rowse[b], [erow, ecol + wv])
                        bb = plsc.load_gather(rowsa[b], [eidx, wv])
                        c = plsc.load_gather(rowsb[b], [eidx, wv])
                        vv[(2 * ww) % 4] = (vv[(2 * ww) % 4]
                                            + unlo(a) * unlo(bb) * unlo(c))
                        vv[(2 * ww + 1) % 4] = (vv[(2 * ww + 1) % 4]
                                                + unhi(a) * unhi(bb) * unhi(c))
                    return tuple(vv)

                v = lax.fori_loop(0, nwblk, h_body,
                                  (zero16, zero16, zero16, zero16))
                val = (v[0] + v[1]) + (v[2] + v[3]) + bfcv
                plsc.addupdate_scatter(acc, [gv, lane], val)
                return 0

            lax.fori_loop(0, _C // 16, grp_body, 0)

    issue_idx(0, 0)
    start_rows(0, 0)
    issue_idx(1, 1)

    def pair_body(p, _):
        for b in (0, 1):
            j = 2 * p + b
            start_rows(j + 1, 1 - b)
            compute(j, b)
            issue_idx(j + 2, b)
        return 0

    lax.fori_loop(0, ntrip // 2, pair_body, 0)

    pltpu.sync_copy(acc, out_part.at[wid])


def kernel(h_ligand, h_pocket, e_lp, e_pl, lp_src, lp_dst, pl_src, pl_dst,
           lp_graph_id, pl_graph_id,
           W_lp_src, b_lp_src, W_lp_dst, b_lp_dst, W_lp_edge, b_lp_edge,
           W_pl_src, b_pl_src, W_pl_dst, b_pl_dst, W_pl_edge, b_pl_edge,
           W_fc_lp, b_fc_lp, W_fc_pl, b_fc_pl):
    f32 = jnp.float32
    i32 = jnp.int32

    n_grid = _N_NODE // _NODE_BLK
    node_spec = pl.BlockSpec((_NODE_BLK, _NODE_F), lambda i: (i, 0))
    nw_spec = pl.BlockSpec((_NODE_F, _HID), lambda i: (0, 0))
    nb_spec = pl.BlockSpec((1, _HID), lambda i: (0, 0))
    nout_spec = pl.BlockSpec((_NODE_BLK, _HID // 2), lambda i: (i, 0))
    hs_lp, hs_pl, hd_lp, hd_pl = pl.pallas_call(
        _node_proj_body,
        grid=(n_grid,),
        in_specs=[node_spec, node_spec, nw_spec, nw_spec, nw_spec, nw_spec,
                  nb_spec, nb_spec, nb_spec, nb_spec],
        out_specs=[nout_spec] * 4,
        out_shape=[jax.ShapeDtypeStruct((_N_NODE, _HID // 2), i32)] * 4,
    )(h_ligand, h_pocket, W_lp_src, W_pl_src, W_lp_dst, W_pl_dst,
      b_lp_src.reshape(1, _HID), b_pl_src.reshape(1, _HID),
      b_lp_dst.reshape(1, _HID), b_pl_dst.reshape(1, _HID))

    er_rows = _E // 8
    eblk = _EDGE_BLK // 8
    e_grid = er_rows // eblk
    edge_spec = pl.BlockSpec((eblk, 128), lambda i: (i, 0))
    ew_spec = pl.BlockSpec((128, 512), lambda i: (0, 0))
    eb_spec = pl.BlockSpec((1, 512), lambda i: (0, 0))
    eout_spec = pl.BlockSpec((eblk, 256), lambda i: (i, 0))
    edge_proj = pl.pallas_call(
        _edge_proj_body,
        grid=(e_grid,),
        in_specs=[edge_spec, ew_spec, eb_spec],
        out_specs=[eout_spec],
        out_shape=[jax.ShapeDtypeStruct((er_rows, 256), i32)],
    )

    def edge_prep(e, W_edge, b_edge, W_fc):
        wprime = W_edge * W_fc.reshape(1, _HID)
        wb = jnp.kron(jnp.eye(8, dtype=f32), wprime)
        bbt = jnp.tile((b_edge * W_fc.reshape(_HID)).reshape(1, _HID), (1, 8))
        return e.reshape(er_rows, 128), wb, bbt

    mesh = plsc.VectorSubcoreMesh(core_axis_name="c", subcore_axis_name="s",
                                  num_cores=_NC)
    sc_dir = pl.kernel(
        _sc_dir,
        mesh=mesh,
        compiler_params=pltpu.CompilerParams(
            use_tc_tiling_on_sc=False, needs_layout_passes=False),
        out_type=jax.ShapeDtypeStruct((_NW, _G, 16), f32),
        scratch_types=(
            [pltpu.VMEM((_C,), i32)] * 6
            + [pltpu.VMEM((_C, _HID // 2), i32)] * 4
            + [pltpu.VMEM((_C // 8, _HID * 4), i32)] * 2
            + [pltpu.VMEM((_G, 16), f32)]
            + [pltpu.VMEM((16,), f32)]
            + [pltpu.SemaphoreType.DMA] * 4
        ),
    )

    (eh_lp,) = edge_proj(*edge_prep(e_lp, W_lp_edge, b_lp_edge, W_fc_lp))
    part_lp = sc_dir(
        hs_lp, hd_lp, eh_lp,
        lp_src.astype(i32), lp_dst.astype(i32), lp_graph_id.astype(i32),
        jnp.broadcast_to(b_fc_lp.astype(f32), (16,)))

    (eh_pl,) = edge_proj(*edge_prep(e_pl, W_pl_edge, b_pl_edge, W_fc_pl))
    part_pl = sc_dir(
        hd_pl, hs_pl, eh_pl,
        pl_src.astype(i32), pl_dst.astype(i32), pl_graph_id.astype(i32),
        jnp.broadcast_to(b_fc_pl.astype(f32), (16,)))

    logit_lp, logit_pl = pl.pallas_call(
        _combine_body,
        out_shape=[jax.ShapeDtypeStruct((_G, 1), f32)] * 2,
    )(part_lp, part_pl)
    return (logit_lp, logit_pl)


# speedup vs baseline: 5.5742x; 1.0319x over previous
import functools

import jax
import jax.numpy as jnp
from jax import lax
from jax.experimental import pallas as pl
from jax.experimental.pallas import tpu as pltpu
from jax.experimental.pallas import tpu_sc as plsc

_N_NODE = 50000
_E = 800000
_NODE_F = 128
_EDGE_F = 16
_HID = 64
_G = 64

_C = 256
_NCHUNK = _E // _C
_NC = 2
_NS = 16
_NW = _NC * _NS

_NODE_BLK = 5000
_EDGE_BLK = 8000


def _pack_pairs(y):
    lo = jax.lax.bitcast_convert_type(
        y[:, :_HID // 2].astype(jnp.bfloat16), jnp.uint16).astype(jnp.uint32)
    hi = jax.lax.bitcast_convert_type(
        y[:, _HID // 2:].astype(jnp.bfloat16), jnp.uint16).astype(jnp.uint32)
    return jax.lax.bitcast_convert_type(lo | (hi << 16), jnp.int32)


def _node_proj_body(hl, hp, wls, wps, wld, wpd, bls, bps, bld, bpd,
                    hs_lp, hs_pl, hd_lp, hd_pl):
    l = hl[...]
    p = hp[...]
    hs_lp[...] = _pack_pairs(jnp.dot(l, wls[...], preferred_element_type=jnp.float32) + bls[...])
    hs_pl[...] = _pack_pairs(jnp.dot(l, wps[...], preferred_element_type=jnp.float32) + bps[...])
    hd_lp[...] = _pack_pairs(jnp.dot(p, wld[...], preferred_element_type=jnp.float32) + bld[...])
    hd_pl[...] = _pack_pairs(jnp.dot(p, wpd[...], preferred_element_type=jnp.float32) + bpd[...])


def _edge_proj_body(er, wb, bbt, out_l):
    y = jnp.dot(er[...], wb[...], preferred_element_type=jnp.float32) + bbt[...]
    for j in range(8):
        out_l[:, j * 32:(j + 1) * 32] = _pack_pairs(y[:, j * 64:(j + 1) * 64])


def _combine_body(plp, ppl, olp, opl):
    a = jnp.sum(plp[...], axis=0)
    b = jnp.sum(ppl[...], axis=0)
    olp[...] = jnp.sum(a, axis=1, keepdims=True)
    opl[...] = jnp.sum(b, axis=1, keepdims=True)


def _sc_dir(taba, tabb, ehtab, src_hbm, dst_hbm, gid_hbm, bfc_hbm,
            out_part,
            idxa0, idxa1, idxb0, idxb1, gidb0, gidb1,
            rowsa0, rowsa1, rowsb0, rowsb1, rowse0, rowse1,
            acc, bfcbuf,
            sidx0, sidx1, srows0, srows1):
    cid = lax.axis_index("c")
    sid = lax.axis_index("s")
    wid = sid * _NC + cid

    ntrip = -(-_NCHUNK // _NW)
    assert ntrip % 2 == 0

    idxa = [idxa0, idxa1]
    idxb = [idxb0, idxb1]
    gidb = [gidb0, gidb1]
    rowsa = [rowsa0, rowsa1]
    rowsb = [rowsb0, rowsb1]
    rowse = [rowse0, rowse1]
    sidx = [sidx0, sidx1]
    srows = [srows0, srows1]

    zero16 = jnp.zeros((16,), jnp.float32)
    for i in range(_G):
        acc[i] = zero16

    lane = lax.iota(jnp.int32, 16)
    nw_words = _HID // 2
    nwblk = 4
    wunroll = nw_words // nwblk

    def unlo(x):
        return plsc.bitcast(jax.lax.shift_left(x, 16), jnp.float32)

    def unhi(x):
        return plsc.bitcast(jnp.bitwise_and(x, jnp.int32(-65536)), jnp.float32)

    pltpu.sync_copy(bfc_hbm, bfcbuf)
    bfcv = bfcbuf[...]

    def idx_copies(ci, b):
        e0 = ci * _C
        return [
            pltpu.make_async_copy(src_hbm.at[pl.ds(e0, _C)], idxa[b], sidx[b]),
            pltpu.make_async_copy(dst_hbm.at[pl.ds(e0, _C)], idxb[b], sidx[b]),
            pltpu.make_async_copy(gid_hbm.at[pl.ds(e0, _C)], gidb[b], sidx[b]),
        ]

    def rows_copies(ci, b):
        cps = []
        for k in range(_C // 128):
            sl = pl.ds(k * 128, 128)
            cps.append(pltpu.make_async_copy(
                taba.at[idxa[b].at[sl]], rowsa[b].at[sl], srows[b]))
            cps.append(pltpu.make_async_copy(
                tabb.at[idxb[b].at[sl]], rowsb[b].at[sl], srows[b]))
        cps.append(pltpu.make_async_copy(
            ehtab.at[pl.ds(ci * (_C // 8), _C // 8)], rowse[b], srows[b]))
        return cps

    def issue_idx(j, b):
        ci = wid + _NW * j

        @pl.when((j < ntrip) & (ci < _NCHUNK))
        def _():
            for cp in idx_copies(ci, b):
                cp.start()

    def start_rows(j, b):
        ci = wid + _NW * j

        @pl.when((j < ntrip) & (ci < _NCHUNK))
        def _():
            for cp in idx_copies(ci, b):
                cp.wait()
            for cp in rows_copies(ci, b):
                cp.start()

    def compute(j, b):
        ci = wid + _NW * j

        @pl.when(ci < _NCHUNK)
        def _():
            for cp in rows_copies(ci, b):
                cp.wait()

            def grp_body(g, _):
                eidx = lane + g * 16
                erow = lax.shift_right_logical(eidx, 3)
                ecol = lax.shift_left(jnp.bitwise_and(eidx, 7), 5)
                gv = gidb[b][pl.ds(g * 16, 16)]

                def h_body(hb, v):
                    vv = list(v)
                    base = hb * wunroll
                    for ww in range(wunroll):
                        wv = jnp.bitwise_and(lane + (base + ww), nw_words - 1)
                        a = plsc.load_gather(rowse[b], [erow, ecol + wv])
                        bb = plsc.load_gather(rowsa[b], [eidx, wv])
                        c = plsc.load_gather(rowsb[b], [eidx, wv])
                        vv[(2 * ww) % 4] = (vv[(2 * ww) % 4]
                                            + unlo(a) * unlo(bb) * unlo(c))
                        vv[(2 * ww + 1) % 4] = (vv[(2 * ww + 1) % 4]
                                                + unhi(a) * unhi(bb) * unhi(c))
                    return tuple(vv)

                v = lax.fori_loop(0, nwblk, h_body,
                                  (zero16, zero16, zero16, zero16))
                val = (v[0] + v[1]) + (v[2] + v[3]) + bfcv
                plsc.addupdate_scatter(acc, [gv, lane], val)
                return 0

            lax.fori_loop(0, _C // 16, grp_body, 0)

    issue_idx(0, 0)
    start_rows(0, 0)
    issue_idx(1, 1)

    def pair_body(p, _):
        for b in (0, 1):
            j = 2 * p + b
            start_rows(j + 1, 1 - b)
            compute(j, b)
            issue_idx(j + 2, b)
        return 0

    lax.fori_loop(0, ntrip // 2, pair_body, 0)

    pltpu.sync_copy(acc, out_part.at[wid])


def kernel(h_ligand, h_pocket, e_lp, e_pl, lp_src, lp_dst, pl_src, pl_dst,
           lp_graph_id, pl_graph_id,
           W_lp_src, b_lp_src, W_lp_dst, b_lp_dst, W_lp_edge, b_lp_edge,
           W_pl_src, b_pl_src, W_pl_dst, b_pl_dst, W_pl_edge, b_pl_edge,
           W_fc_lp, b_fc_lp, W_fc_pl, b_fc_pl):
    f32 = jnp.float32
    i32 = jnp.int32

    n_grid = _N_NODE // _NODE_BLK
    node_spec = pl.BlockSpec((_NODE_BLK, _NODE_F), lambda i: (i, 0))
    nw_spec = pl.BlockSpec((_NODE_F, _HID), lambda i: (0, 0))
    nb_spec = pl.BlockSpec((1, _HID), lambda i: (0, 0))
    nout_spec = pl.BlockSpec((_NODE_BLK, _HID // 2), lambda i: (i, 0))
    hs_lp, hs_pl, hd_lp, hd_pl = pl.pallas_call(
        _node_proj_body,
        grid=(n_grid,),
        in_specs=[node_spec, node_spec, nw_spec, nw_spec, nw_spec, nw_spec,
                  nb_spec, nb_spec, nb_spec, nb_spec],
        out_specs=[nout_spec] * 4,
        out_shape=[jax.ShapeDtypeStruct((_N_NODE, _HID // 2), i32)] * 4,
    )(h_ligand, h_pocket, W_lp_src, W_pl_src, W_lp_dst, W_pl_dst,
      b_lp_src.reshape(1, _HID), b_pl_src.reshape(1, _HID),
      b_lp_dst.reshape(1, _HID), b_pl_dst.reshape(1, _HID))

    er_rows = _E // 8
    eblk = 2000
    e_grid = er_rows // eblk
    edge_spec = pl.BlockSpec((eblk, 128), lambda i: (i, 0))
    ew_spec = pl.BlockSpec((128, 512), lambda i: (0, 0))
    eb_spec = pl.BlockSpec((1, 512), lambda i: (0, 0))
    eout_spec = pl.BlockSpec((eblk, 256), lambda i: (i, 0))
    edge_proj = pl.pallas_call(
        _edge_proj_body,
        grid=(e_grid,),
        in_specs=[edge_spec, ew_spec, eb_spec],
        out_specs=[eout_spec],
        out_shape=[jax.ShapeDtypeStruct((er_rows, 256), i32)],
    )

    def edge_prep(e, W_edge, b_edge, W_fc):
        wprime = W_edge * W_fc.reshape(1, _HID)
        wb = jnp.kron(jnp.eye(8, dtype=f32), wprime)
        bbt = jnp.tile((b_edge * W_fc.reshape(_HID)).reshape(1, _HID), (1, 8))
        return e.reshape(er_rows, 128), wb, bbt

    mesh = plsc.VectorSubcoreMesh(core_axis_name="c", subcore_axis_name="s",
                                  num_cores=_NC)
    sc_dir = pl.kernel(
        _sc_dir,
        mesh=mesh,
        compiler_params=pltpu.CompilerParams(
            use_tc_tiling_on_sc=False, needs_layout_passes=False),
        out_type=jax.ShapeDtypeStruct((_NW, _G, 16), f32),
        scratch_types=(
            [pltpu.VMEM((_C,), i32)] * 6
            + [pltpu.VMEM((_C, _HID // 2), i32)] * 4
            + [pltpu.VMEM((_C // 8, _HID * 4), i32)] * 2
            + [pltpu.VMEM((_G, 16), f32)]
            + [pltpu.VMEM((16,), f32)]
            + [pltpu.SemaphoreType.DMA] * 4
        ),
    )

    (eh_lp,) = edge_proj(*edge_prep(e_lp, W_lp_edge, b_lp_edge, W_fc_lp))
    part_lp = sc_dir(
        hs_lp, hd_lp, eh_lp,
        lp_src.astype(i32), lp_dst.astype(i32), lp_graph_id.astype(i32),
        jnp.broadcast_to(b_fc_lp.astype(f32), (16,)))

    (eh_pl,) = edge_proj(*edge_prep(e_pl, W_pl_edge, b_pl_edge, W_fc_pl))
    part_pl = sc_dir(
        hd_pl, hs_pl, eh_pl,
        pl_src.astype(i32), pl_dst.astype(i32), pl_graph_id.astype(i32),
        jnp.broadcast_to(b_fc_pl.astype(f32), (16,)))

    logit_lp, logit_pl = pl.pallas_call(
        _combine_body,
        out_shape=[jax.ShapeDtypeStruct((_G, 1), f32)] * 2,
    )(part_lp, part_pl)
    return (logit_lp, logit_pl)
